# baseline (speedup 1.0000x reference)
import functools
import jax
import jax.numpy as jnp
from jax import lax
from jax.experimental import pallas as pl
from jax.experimental.pallas import tpu as pltpu
from jax.experimental.pallas import tpu_sc as plsc

_L = 16
_CH = 16


def _sc_add(x_hbm, emb_hbm, out_hbm, xb, eb, sx0, sx1, se0, se1, ss0, ss1):
    batch, seq_len, d_model = x_hbm.shape
    n_workers = 32
    s_per_w = seq_len // n_workers
    n_ch = s_per_w // _CH
    n_it = n_ch * batch

    wid = lax.axis_index("s") * 2 + lax.axis_index("c")
    s_base = wid * s_per_w

    semx = [sx0, sx1]
    seme = [se0, se1]
    sems = [ss0, ss1]

    def s0_of(c):
        return s_base + c * _CH

    def x_slice(i):
        return (i % batch, pl.ds(s0_of(i // batch), _CH))

    ecp = [None, None]
    xcp = [None, None]
    scp = [None, None]
    ecp[0] = pltpu.async_copy(emb_hbm.at[pl.ds(s0_of(0), _CH)], eb.at[0], seme[0])
    b0, sl0 = x_slice(0)
    xcp[0] = pltpu.async_copy(x_hbm.at[b0, sl0], xb.at[0], semx[0])

    for c in range(n_ch):
        ec = c % 2
        if c + 1 < n_ch:
            ecp[(c + 1) % 2] = pltpu.async_copy(
                emb_hbm.at[pl.ds(s0_of(c + 1), _CH)], eb.at[(c + 1) % 2],
                seme[(c + 1) % 2])
        ecp[ec].wait()
        for b in range(batch):
            i = c * batch + b
            slot = i % 2
            nxt = (i + 1) % 2
            if i + 1 < n_it:
                if scp[nxt] is not None:
                    scp[nxt].wait()
                    scp[nxt] = None
                bn, sln = x_slice(i + 1)
                xcp[nxt] = pltpu.async_copy(x_hbm.at[bn, sln], xb.at[nxt],
                                            semx[nxt])
            xcp[slot].wait()

            def row_body(r, _):
                def col_body(g, _):
                    for u in range(8):
                        sl = pl.ds((g * 8 + u) * _L, _L)
                        xb[slot, r, sl] = xb[slot, r, sl] + eb[ec, r, sl]
                    return 0

                return lax.fori_loop(0, d_model // (8 * _L), col_body, 0)

            lax.fori_loop(0, _CH, row_body, 0)
            if scp[slot] is not None:
                scp[slot].wait()
                scp[slot] = None
            bi, sli = x_slice(i)
            scp[slot] = pltpu.async_copy(xb.at[slot], out_hbm.at[bi, sli],
                                         sems[slot])
    for k in range(2):
        if scp[k] is not None:
            scp[k].wait()


def kernel(x, emb_weight):
    batch, seq_len, d_model = x.shape
    run = functools.partial(
        pl.kernel,
        mesh=plsc.VectorSubcoreMesh(core_axis_name="c", subcore_axis_name="s"),
        out_type=jax.ShapeDtypeStruct((batch, seq_len, d_model), x.dtype),
        scratch_types=[
            pltpu.VMEM((2, _CH, d_model), jnp.float32),
            pltpu.VMEM((2, _CH, d_model), jnp.float32),
            pltpu.SemaphoreType.DMA,
            pltpu.SemaphoreType.DMA,
            pltpu.SemaphoreType.DMA,
            pltpu.SemaphoreType.DMA,
            pltpu.SemaphoreType.DMA,
            pltpu.SemaphoreType.DMA,
        ],
    )(_sc_add)
    return run(x, emb_weight)


# speedup vs baseline: 4.7773x; 4.7773x over previous
import jax
import jax.numpy as jnp
from jax.experimental import pallas as pl
from jax.experimental.pallas import tpu as pltpu


def _add_kernel(x_ref, emb_ref, o_ref):
    o_ref[...] = x_ref[...] + emb_ref[...]


def kernel(x, emb_weight):
    batch, seq_len, d_model = x.shape

    s_blk = 1024
    while seq_len % s_blk:
        s_blk //= 2
    num_s = seq_len // s_blk

    x2 = x.reshape(batch * seq_len, d_model)
    out = pl.pallas_call(
        _add_kernel,
        grid=(num_s, batch),
        in_specs=[
            pl.BlockSpec((s_blk, d_model), lambda s, b: (b * num_s + s, 0)),
            pl.BlockSpec((s_blk, d_model), lambda s, b: (s, 0)),
        ],
        out_specs=pl.BlockSpec((s_blk, d_model), lambda s, b: (b * num_s + s, 0)),
        out_shape=jax.ShapeDtypeStruct((batch * seq_len, d_model), x.dtype),
        compiler_params=pltpu.CompilerParams(
            dimension_semantics=("arbitrary", "arbitrary"),
        ),
    )(x2, emb_weight)
    return out.reshape(batch, seq_len, d_model)


# speedup vs baseline: 5.0678x; 1.0608x over previous
import jax
import jax.numpy as jnp
from jax.experimental import pallas as pl
from jax.experimental.pallas import tpu as pltpu


def _add_kernel(x_ref, emb_ref, o_ref):
    o_ref[...] = x_ref[...] + emb_ref[...]


def kernel(x, emb_weight):
    batch, seq_len, d_model = x.shape

    s_blk = 2048
    while seq_len % s_blk:
        s_blk //= 2
    num_s = seq_len // s_blk

    x2 = x.reshape(batch * seq_len, d_model)
    out = pl.pallas_call(
        _add_kernel,
        grid=(num_s, batch),
        in_specs=[
            pl.BlockSpec((s_blk, d_model), lambda s, b: (b * num_s + s, 0)),
            pl.BlockSpec((s_blk, d_model), lambda s, b: (s, 0)),
        ],
        out_specs=pl.BlockSpec((s_blk, d_model), lambda s, b: (b * num_s + s, 0)),
        out_shape=jax.ShapeDtypeStruct((batch * seq_len, d_model), x.dtype),
        compiler_params=pltpu.CompilerParams(
            dimension_semantics=("arbitrary", "arbitrary"),
        ),
    )(x2, emb_weight)
    return out.reshape(batch, seq_len, d_model)
